# Initial kernel scaffold; baseline (speedup 1.0000x reference)
import functools
import math

import jax
import jax.numpy as jnp
from jax import lax
from jax.experimental import pallas as pl
from jax.experimental.pallas import tpu as pltpu
from jax.experimental.pallas import tpu_sc as plsc

N = 10000
NE = 10000
NNZ = 320000
D = 128
ALPHA = 0.1
LAMDA = 0.5

NC = 2
NS = 16
NW = NC * NS
NP = 10240
RPT = NP // NS
K = 80
PER_W = NNZ // NW
CHUNKS = PER_W // K

_mesh = plsc.VectorSubcoreMesh(core_axis_name="c", subcore_axis_name="s")
_f32 = jnp.float32


def _sc_phase_body(with_count, *refs):
    if with_count:
        (src, gidx, sidx, z128, z16, ones_h,
         out, cnt_out,
         acc, cnt, ig, isc, rows, ones_v, sem) = refs
    else:
        (src, gidx, sidx, z128,
         out,
         acc, ig, isc, rows, sem) = refs
    c = lax.axis_index("c")
    s = lax.axis_index("s")
    wid = s * NC + c
    row0 = s * RPT
    pltpu.sync_copy(z128.at[pl.ds(row0, RPT)], acc.at[pl.ds(row0, RPT)])
    if with_count:
        pltpu.sync_copy(z16.at[pl.ds(row0, RPT)], cnt.at[pl.ds(row0, RPT)])
        pltpu.sync_copy(ones_h, ones_v)
    plsc.subcore_barrier()

    base0 = wid * PER_W

    def chunk(j, carry):
        base = pl.multiple_of(base0 + j * K, 8)
        pltpu.sync_copy(gidx.at[pl.ds(base, K)], ig)
        pltpu.sync_copy(sidx.at[pl.ds(base, K)], isc)
        pltpu.async_copy(src.at[ig], rows, sem).wait()
        pltpu.sync_copy(rows, acc.at[isc], add=True)
        if with_count:
            pltpu.sync_copy(ones_v, cnt.at[isc], add=True)
        return carry

    lax.fori_loop(0, CHUNKS, chunk, 0)
    plsc.subcore_barrier()
    obase = pl.multiple_of(c * NP + row0, 8)
    pltpu.sync_copy(acc.at[pl.ds(row0, RPT)], out.at[pl.ds(obase, RPT)])
    if with_count:
        pltpu.sync_copy(cnt.at[pl.ds(row0, RPT)], cnt_out.at[pl.ds(obase, RPT)])


def _make_sc_phase(with_count):
    if with_count:
        out_type = (jax.ShapeDtypeStruct((2 * NP, D), _f32),
                    jax.ShapeDtypeStruct((2 * NP, 16), _f32))
        scratch = [
            pltpu.VMEM_SHARED((NP, D), _f32),
            pltpu.VMEM_SHARED((NP, 16), _f32),
            pltpu.VMEM((K,), jnp.int32),
            pltpu.VMEM((K,), jnp.int32),
            pltpu.VMEM((K, D), _f32),
            pltpu.VMEM((K, 16), _f32),
            pltpu.SemaphoreType.DMA,
        ]
    else:
        out_type = jax.ShapeDtypeStruct((2 * NP, D), _f32)
        scratch = [
            pltpu.VMEM_SHARED((NP, D), _f32),
            pltpu.VMEM((K,), jnp.int32),
            pltpu.VMEM((K,), jnp.int32),
            pltpu.VMEM((K, D), _f32),
            pltpu.SemaphoreType.DMA,
        ]
    return pl.kernel(
        functools.partial(_sc_phase_body, with_count),
        out_type=out_type,
        mesh=_mesh,
        scratch_types=scratch,
    )


_sc_phase_cnt = _make_sc_phase(True)
_sc_phase = _make_sc_phase(False)


BR = 1024
GRID = NP // BR


def _linear_relu_body(x, w, b, o):
    o[...] = jax.nn.relu(
        jnp.dot(x[...], w[...], preferred_element_type=_f32) + b[...])


def _tc_linear_relu(x, wT, b):
    return pl.pallas_call(
        _linear_relu_body,
        grid=(GRID,),
        in_specs=[
            pl.BlockSpec((BR, D), lambda i: (i, 0)),
            pl.BlockSpec((D, D), lambda i: (0, 0)),
            pl.BlockSpec((1, D), lambda i: (0, 0)),
        ],
        out_specs=pl.BlockSpec((BR, D), lambda i: (i, 0)),
        out_shape=jax.ShapeDtypeStruct((NP, D), _f32),
    )(x, wT, b)


def _mean_body(p, cnt, o):
    s = p[0] + p[1]
    c = cnt[0, :, 0:1] + cnt[1, :, 0:1]
    o[...] = s / jnp.maximum(c, 1.0)


def _tc_mean(parts, cnt_parts):
    return pl.pallas_call(
        _mean_body,
        grid=(GRID,),
        in_specs=[
            pl.BlockSpec((2, BR, D), lambda i: (0, i, 0)),
            pl.BlockSpec((2, BR, 16), lambda i: (0, i, 0)),
        ],
        out_specs=pl.BlockSpec((BR, D), lambda i: (i, 0)),
        out_shape=jax.ShapeDtypeStruct((NP, D), _f32),
    )(parts, cnt_parts)


def _layer_body(beta, p, h0, wT, o):
    xv = p[0] + p[1]
    xi = (1.0 - ALPHA) * xv + ALPHA * h0[...]
    o[...] = jax.nn.relu(
        (1.0 - beta) * xi
        + beta * jnp.dot(xi, wT[...], preferred_element_type=_f32))


def _tc_layer(parts, h0, wT, beta):
    return pl.pallas_call(
        functools.partial(_layer_body, beta),
        grid=(GRID,),
        in_specs=[
            pl.BlockSpec((2, BR, D), lambda i: (0, i, 0)),
            pl.BlockSpec((BR, D), lambda i: (i, 0)),
            pl.BlockSpec((D, D), lambda i: (0, 0)),
        ],
        out_specs=pl.BlockSpec((BR, D), lambda i: (i, 0)),
        out_shape=jax.ShapeDtypeStruct((NP, D), _f32),
    )(parts, h0, wT)


def _final_body(ncls, h, wT, b, o):
    z = jnp.dot(h[...], wT[...], preferred_element_type=_f32) + b[...]
    col = lax.broadcasted_iota(jnp.int32, (BR, D), 1)
    valid = col < ncls
    zm = jnp.where(valid, z, -1e30)
    m = jnp.max(zm, axis=1, keepdims=True)
    e = jnp.where(valid, jnp.exp(z - m), 0.0)
    ssum = jnp.sum(e, axis=1, keepdims=True)
    o[...] = z - m - jnp.log(ssum)


def _tc_final(h, wT, b, ncls):
    return pl.pallas_call(
        functools.partial(_final_body, ncls),
        grid=(GRID,),
        in_specs=[
            pl.BlockSpec((BR, D), lambda i: (i, 0)),
            pl.BlockSpec((D, D), lambda i: (0, 0)),
            pl.BlockSpec((1, D), lambda i: (0, 0)),
        ],
        out_specs=pl.BlockSpec((BR, D), lambda i: (i, 0)),
        out_shape=jax.ShapeDtypeStruct((NP, D), _f32),
    )(h, wT, b)


def kernel(x, V, E, W0, b0, Wc0, Wc1, Wout, bout):
    V = V.astype(jnp.int32)
    E = E.astype(jnp.int32)
    ncls = Wout.shape[0]

    xp = jnp.zeros((NP, D), _f32).at[:N].set(x)
    z128 = jnp.zeros((NP, D), _f32)
    z16 = jnp.zeros((NP, 16), _f32)
    ones = jnp.ones((K, 16), _f32)

    h = _tc_linear_relu(xp, W0.T, b0[None, :])
    h0 = h

    woutT = jnp.zeros((D, D), _f32).at[:, :ncls].set(Wout.T)
    bout_p = jnp.zeros((1, D), _f32).at[0, :ncls].set(bout)

    cnt_parts = None
    for i, Wc in enumerate([Wc0, Wc1]):
        beta = math.log(LAMDA / (i + 1) + 1.0)
        if cnt_parts is None:
            pe, cnt_flat = _sc_phase_cnt(h, V, E, z128, z16, ones)
            cnt_parts = cnt_flat.reshape(2, NP, 16)
        else:
            pe = _sc_phase(h, V, E, z128)
        xe = _tc_mean(pe.reshape(2, NP, D), cnt_parts)
        pv = _sc_phase(xe, E, V, z128)
        h = _tc_layer(pv.reshape(2, NP, D), h0, Wc.T, beta)

    out = _tc_final(h, woutT, bout_p, ncls)
    return out[:N, :ncls]


# speedup vs baseline: 11.5763x; 11.5763x over previous
import functools
import math

import jax
import jax.numpy as jnp
from jax import lax
from jax.experimental import pallas as pl
from jax.experimental.pallas import tpu as pltpu
from jax.experimental.pallas import tpu_sc as plsc

N = 10000
NE = 10000
NNZ = 320000
D = 128
ALPHA = 0.1
LAMDA = 0.5

NC = 2
NS = 16
NW = NC * NS
NP = 10240
RPT = NP // NS
K = 128
NNZ_PAD = 327680
NCHUNK = NNZ_PAD // K
CPW = NCHUNK // NW
NBUF = 2

_mesh = plsc.VectorSubcoreMesh(core_axis_name="c", subcore_axis_name="s")
_f32 = jnp.float32


def _sc_phase_body(with_count, *refs):
    if with_count:
        (src, gidx, sidx, z128, z1, ones1h,
         out, cnt_out,
         acc, acc1, ig0, ig1, is0, is1, r0, r1, ones1,
         sgi0, sgi1, ssi0, ssi1, sr0, sr1, sw0, sw1, sc0, sc1) = refs
        sc_ = (sc0, sc1)
    else:
        (src, gidx, sidx, z128,
         out,
         acc, ig0, ig1, is0, is1, r0, r1,
         sgi0, sgi1, ssi0, ssi1, sr0, sr1, sw0, sw1) = refs
    ig = (ig0, ig1)
    isc = (is0, is1)
    rows = (r0, r1)
    sgi = (sgi0, sgi1)
    ssi = (ssi0, ssi1)
    sr = (sr0, sr1)
    sw = (sw0, sw1)
    c = lax.axis_index("c")
    s = lax.axis_index("s")
    wid = s * NC + c
    row0 = s * RPT
    pltpu.sync_copy(z128.at[pl.ds(row0, RPT)], acc.at[pl.ds(row0, RPT)])
    if with_count:
        pltpu.sync_copy(z1.at[pl.ds(row0, RPT)], acc1.at[pl.ds(row0, RPT)])
        pltpu.sync_copy(ones1h, ones1)
    plsc.subcore_barrier()

    base0 = pl.multiple_of(wid * CPW * K, 8)

    def idx_slice(h, j):
        return h.at[pl.ds(base0 + j * K, K)]

    for b in range(NBUF):
        pltpu.async_copy(idx_slice(gidx, b), ig[b], sgi[b])
        pltpu.async_copy(idx_slice(sidx, b), isc[b], ssi[b])
    for b in range(NBUF):
        pltpu.make_async_copy(idx_slice(gidx, b), ig[b], sgi[b]).wait()
        pltpu.async_copy(src.at[ig[b]], rows[b], sr[b])

    def body(t, carry):
        for b in range(NBUF):
            j = t * NBUF + b
            pltpu.make_async_copy(src.at[ig[b]], rows[b], sr[b]).wait()
            pltpu.async_copy(idx_slice(gidx, j + NBUF), ig[b], sgi[b])
            pltpu.make_async_copy(idx_slice(sidx, j), isc[b], ssi[b]).wait()
            pltpu.async_copy(rows[b], acc.at[isc[b]], sw[b], add=True)
            if with_count:
                pltpu.async_copy(ones1, acc1.at[isc[b]], sc_[b], add=True)
            pltpu.make_async_copy(rows[b], acc.at[isc[b]], sw[b]).wait()
            if with_count:
                pltpu.make_async_copy(ones1, acc1.at[isc[b]], sc_[b]).wait()
            pltpu.async_copy(idx_slice(sidx, j + NBUF), isc[b], ssi[b])
            pltpu.make_async_copy(idx_slice(gidx, j + NBUF), ig[b], sgi[b]).wait()
            pltpu.async_copy(src.at[ig[b]], rows[b], sr[b])
        return carry

    lax.fori_loop(0, CPW // NBUF - 1, body, 0)
    for b in range(NBUF):
        j = CPW - NBUF + b
        pltpu.make_async_copy(src.at[ig[b]], rows[b], sr[b]).wait()
        pltpu.make_async_copy(idx_slice(sidx, j), isc[b], ssi[b]).wait()
        pltpu.async_copy(rows[b], acc.at[isc[b]], sw[b], add=True)
        if with_count:
            pltpu.async_copy(ones1, acc1.at[isc[b]], sc_[b], add=True)
    for b in range(NBUF):
        pltpu.make_async_copy(rows[b], acc.at[isc[b]], sw[b]).wait()
        if with_count:
            pltpu.make_async_copy(ones1, acc1.at[isc[b]], sc_[b]).wait()
    plsc.subcore_barrier()
    obase = pl.multiple_of(c * NP + row0, 8)
    pltpu.sync_copy(acc.at[pl.ds(row0, RPT)], out.at[pl.ds(obase, RPT)])
    if with_count:
        pltpu.sync_copy(acc1.at[pl.ds(row0, RPT)], cnt_out.at[pl.ds(obase, RPT)])


def _make_phase(with_count):
    if with_count:
        out_type = (jax.ShapeDtypeStruct((2 * NP, D), _f32),
                    jax.ShapeDtypeStruct((2 * NP,), _f32))
        scratch = [
            pltpu.VMEM_SHARED((NP, D), _f32),
            pltpu.VMEM_SHARED((NP,), _f32),
            pltpu.VMEM((K,), jnp.int32),
            pltpu.VMEM((K,), jnp.int32),
            pltpu.VMEM((K,), jnp.int32),
            pltpu.VMEM((K,), jnp.int32),
            pltpu.VMEM((K, D), _f32),
            pltpu.VMEM((K, D), _f32),
            pltpu.VMEM((K,), _f32),
        ] + [pltpu.SemaphoreType.DMA] * 10
    else:
        out_type = jax.ShapeDtypeStruct((2 * NP, D), _f32)
        scratch = [
            pltpu.VMEM_SHARED((NP, D), _f32),
            pltpu.VMEM((K,), jnp.int32),
            pltpu.VMEM((K,), jnp.int32),
            pltpu.VMEM((K,), jnp.int32),
            pltpu.VMEM((K,), jnp.int32),
            pltpu.VMEM((K, D), _f32),
            pltpu.VMEM((K, D), _f32),
        ] + [pltpu.SemaphoreType.DMA] * 8
    return pl.kernel(
        functools.partial(_sc_phase_body, with_count),
        out_type=out_type,
        mesh=_mesh,
        scratch_types=scratch,
    )


_sc_phase_cnt = _make_phase(True)
_sc_phase = _make_phase(False)


BR = 1024
GRID = NP // BR


def _linear_relu_body(x, w, b, o):
    o[...] = jax.nn.relu(
        jnp.dot(x[...], w[...], preferred_element_type=_f32) + b[...])


def _tc_linear_relu(x, wT, b):
    return pl.pallas_call(
        _linear_relu_body,
        grid=(GRID,),
        in_specs=[
            pl.BlockSpec((BR, D), lambda i: (i, 0)),
            pl.BlockSpec((D, D), lambda i: (0, 0)),
            pl.BlockSpec((1, D), lambda i: (0, 0)),
        ],
        out_specs=pl.BlockSpec((BR, D), lambda i: (i, 0)),
        out_shape=jax.ShapeDtypeStruct((NP, D), _f32),
    )(x, wT, b)


def _mean_body(p, cnt, o):
    s = p[0] + p[1]
    c = cnt[0] + cnt[1]
    o[...] = s / jnp.maximum(c, 1.0)


def _tc_mean(parts, cnt_parts):
    return pl.pallas_call(
        _mean_body,
        grid=(GRID,),
        in_specs=[
            pl.BlockSpec((2, BR, D), lambda i: (0, i, 0)),
            pl.BlockSpec((2, BR, 1), lambda i: (0, i, 0)),
        ],
        out_specs=pl.BlockSpec((BR, D), lambda i: (i, 0)),
        out_shape=jax.ShapeDtypeStruct((NP, D), _f32),
    )(parts, cnt_parts)


def _layer_body(beta, p, h0, wT, o):
    xv = p[0] + p[1]
    xi = (1.0 - ALPHA) * xv + ALPHA * h0[...]
    o[...] = jax.nn.relu(
        (1.0 - beta) * xi
        + beta * jnp.dot(xi, wT[...], preferred_element_type=_f32))


def _tc_layer(parts, h0, wT, beta):
    return pl.pallas_call(
        functools.partial(_layer_body, beta),
        grid=(GRID,),
        in_specs=[
            pl.BlockSpec((2, BR, D), lambda i: (0, i, 0)),
            pl.BlockSpec((BR, D), lambda i: (i, 0)),
            pl.BlockSpec((D, D), lambda i: (0, 0)),
        ],
        out_specs=pl.BlockSpec((BR, D), lambda i: (i, 0)),
        out_shape=jax.ShapeDtypeStruct((NP, D), _f32),
    )(parts, h0, wT)


def _layer_final_body(beta, ncls, p, h0, wT, wo, b, o):
    xv = p[0] + p[1]
    xi = (1.0 - ALPHA) * xv + ALPHA * h0[...]
    h = jax.nn.relu(
        (1.0 - beta) * xi
        + beta * jnp.dot(xi, wT[...], preferred_element_type=_f32))
    z = jnp.dot(h, wo[...], preferred_element_type=_f32) + b[...]
    col = lax.broadcasted_iota(jnp.int32, (BR, D), 1)
    valid = col < ncls
    zm = jnp.where(valid, z, -1e30)
    m = jnp.max(zm, axis=1, keepdims=True)
    e = jnp.where(valid, jnp.exp(z - m), 0.0)
    ssum = jnp.sum(e, axis=1, keepdims=True)
    o[...] = z - m - jnp.log(ssum)


def _tc_layer_final(parts, h0, wT, beta, wo, b, ncls):
    return pl.pallas_call(
        functools.partial(_layer_final_body, beta, ncls),
        grid=(GRID,),
        in_specs=[
            pl.BlockSpec((2, BR, D), lambda i: (0, i, 0)),
            pl.BlockSpec((BR, D), lambda i: (i, 0)),
            pl.BlockSpec((D, D), lambda i: (0, 0)),
            pl.BlockSpec((D, D), lambda i: (0, 0)),
            pl.BlockSpec((1, D), lambda i: (0, 0)),
        ],
        out_specs=pl.BlockSpec((BR, D), lambda i: (i, 0)),
        out_shape=jax.ShapeDtypeStruct((NP, D), _f32),
    )(parts, h0, wT, wo, b)


def kernel(x, V, E, W0, b0, Wc0, Wc1, Wout, bout):
    V = V.astype(jnp.int32)
    E = E.astype(jnp.int32)
    ncls = Wout.shape[0]

    xp = jnp.zeros((NP, D), _f32).at[:N].set(x)
    z128 = jnp.zeros((NP, D), _f32)
    z1 = jnp.zeros((NP,), _f32)
    ones1 = jnp.ones((K,), _f32)

    pad = (N + (jnp.arange(NNZ_PAD - NNZ, dtype=jnp.int32) % (NP - N)))
    gp = jnp.concatenate([V, pad])
    sp = jnp.concatenate([E, pad])

    h = _tc_linear_relu(xp, W0.T, b0[None, :])
    h0 = h

    woutT = jnp.zeros((D, D), _f32).at[:, :ncls].set(Wout.T)
    bout_p = jnp.zeros((1, D), _f32).at[0, :ncls].set(bout)

    beta0 = math.log(LAMDA + 1.0)
    beta1 = math.log(LAMDA / 2.0 + 1.0)

    pe, cnt_flat = _sc_phase_cnt(h, gp, sp, z128, z1, ones1)
    cnt_parts = cnt_flat.reshape(2, NP, 1)
    xe = _tc_mean(pe.reshape(2, NP, D), cnt_parts)
    pv = _sc_phase(xe, sp, gp, z128)
    h = _tc_layer(pv.reshape(2, NP, D), h0, Wc0.T, beta0)

    pe = _sc_phase(h, gp, sp, z128)
    xe = _tc_mean(pe.reshape(2, NP, D), cnt_parts)
    pv = _sc_phase(xe, sp, gp, z128)
    out = _tc_layer_final(pv.reshape(2, NP, D), h0, Wc1.T, beta1,
                          woutT, bout_p, ncls)
    return out[:N, :ncls]


# speedup vs baseline: 12.7841x; 1.1043x over previous
import functools
import math

import jax
import jax.numpy as jnp
from jax import lax
from jax.experimental import pallas as pl
from jax.experimental.pallas import tpu as pltpu
from jax.experimental.pallas import tpu_sc as plsc

N = 10000
NE = 10000
NNZ = 320000
D = 128
ALPHA = 0.1
LAMDA = 0.5

NC = 2
NS = 16
NW = NC * NS
NP = 10240
RPT = NP // NS
K = 112
NNZ_PAD = 322560
NCHUNK = NNZ_PAD // K
CPW = NCHUNK // NW
NBUF = 3

_mesh = plsc.VectorSubcoreMesh(core_axis_name="c", subcore_axis_name="s")
_f32 = jnp.float32


def _sc_phase_body(with_count, *refs):
    if with_count:
        src, gidx, sidx, z128, z1, ones1h, out, cnt_out = refs[:8]
        acc, acc1 = refs[8], refs[9]
        rest, ptr = refs, 10
    else:
        src, gidx, sidx, z128, out = refs[:5]
        acc = refs[5]
        rest, ptr = refs, 6
    ig = rest[ptr:ptr + NBUF]; ptr += NBUF
    isc = rest[ptr:ptr + NBUF]; ptr += NBUF
    rows = rest[ptr:ptr + NBUF]; ptr += NBUF
    if with_count:
        ones1 = rest[ptr]; ptr += 1
    sgi = rest[ptr:ptr + NBUF]; ptr += NBUF
    ssi = rest[ptr:ptr + NBUF]; ptr += NBUF
    sr = rest[ptr:ptr + NBUF]; ptr += NBUF
    sw = rest[ptr:ptr + NBUF]; ptr += NBUF
    if with_count:
        sc_ = rest[ptr:ptr + NBUF]
    c = lax.axis_index("c")
    s = lax.axis_index("s")
    wid = s * NC + c
    row0 = s * RPT
    pltpu.sync_copy(z128.at[pl.ds(row0, RPT)], acc.at[pl.ds(row0, RPT)])
    if with_count:
        pltpu.sync_copy(z1.at[pl.ds(row0, RPT)], acc1.at[pl.ds(row0, RPT)])
        pltpu.sync_copy(ones1h, ones1)
    plsc.subcore_barrier()

    base0 = pl.multiple_of(wid * CPW * K, 8)

    def idx_slice(h, j):
        return h.at[pl.ds(base0 + j * K, K)]

    for b in range(NBUF):
        pltpu.async_copy(idx_slice(gidx, b), ig[b], sgi[b])
        pltpu.async_copy(idx_slice(sidx, b), isc[b], ssi[b])
    for b in range(NBUF):
        pltpu.make_async_copy(idx_slice(gidx, b), ig[b], sgi[b]).wait()
        pltpu.async_copy(src.at[ig[b]], rows[b], sr[b])

    def body(t, carry):
        for b in range(NBUF):
            j = t * NBUF + b
            pltpu.make_async_copy(src.at[ig[b]], rows[b], sr[b]).wait()
            pltpu.async_copy(idx_slice(gidx, j + NBUF), ig[b], sgi[b])
            pltpu.make_async_copy(idx_slice(sidx, j), isc[b], ssi[b]).wait()
            pltpu.async_copy(rows[b], acc.at[isc[b]], sw[b], add=True)
            if with_count:
                pltpu.async_copy(ones1, acc1.at[isc[b]], sc_[b], add=True)
            pltpu.make_async_copy(rows[b], acc.at[isc[b]], sw[b]).wait()
            if with_count:
                pltpu.make_async_copy(ones1, acc1.at[isc[b]], sc_[b]).wait()
            pltpu.async_copy(idx_slice(sidx, j + NBUF), isc[b], ssi[b])
            pltpu.make_async_copy(idx_slice(gidx, j + NBUF), ig[b], sgi[b]).wait()
            pltpu.async_copy(src.at[ig[b]], rows[b], sr[b])
        return carry

    lax.fori_loop(0, CPW // NBUF - 1, body, 0)
    for b in range(NBUF):
        j = CPW - NBUF + b
        pltpu.make_async_copy(src.at[ig[b]], rows[b], sr[b]).wait()
        pltpu.make_async_copy(idx_slice(sidx, j), isc[b], ssi[b]).wait()
        pltpu.async_copy(rows[b], acc.at[isc[b]], sw[b], add=True)
        if with_count:
            pltpu.async_copy(ones1, acc1.at[isc[b]], sc_[b], add=True)
    for b in range(NBUF):
        pltpu.make_async_copy(rows[b], acc.at[isc[b]], sw[b]).wait()
        if with_count:
            pltpu.make_async_copy(ones1, acc1.at[isc[b]], sc_[b]).wait()
    plsc.subcore_barrier()
    obase = pl.multiple_of(c * NP + row0, 8)
    pltpu.sync_copy(acc.at[pl.ds(row0, RPT)], out.at[pl.ds(obase, RPT)])
    if with_count:
        pltpu.sync_copy(acc1.at[pl.ds(row0, RPT)], cnt_out.at[pl.ds(obase, RPT)])


def _make_phase(with_count):
    idx_bufs = [pltpu.VMEM((K,), jnp.int32)] * (2 * NBUF)
    row_bufs = [pltpu.VMEM((K, D), _f32)] * NBUF
    if with_count:
        out_type = (jax.ShapeDtypeStruct((2 * NP, D), _f32),
                    jax.ShapeDtypeStruct((2 * NP,), _f32))
        scratch = ([pltpu.VMEM_SHARED((NP, D), _f32),
                    pltpu.VMEM_SHARED((NP,), _f32)]
                   + idx_bufs + row_bufs + [pltpu.VMEM((K,), _f32)]
                   + [pltpu.SemaphoreType.DMA] * (5 * NBUF))
    else:
        out_type = jax.ShapeDtypeStruct((2 * NP, D), _f32)
        scratch = ([pltpu.VMEM_SHARED((NP, D), _f32)]
                   + idx_bufs + row_bufs
                   + [pltpu.SemaphoreType.DMA] * (4 * NBUF))
    return pl.kernel(
        functools.partial(_sc_phase_body, with_count),
        out_type=out_type,
        mesh=_mesh,
        scratch_types=scratch,
    )


_sc_phase_cnt = _make_phase(True)
_sc_phase = _make_phase(False)


BR = 1024
GRID = NP // BR


def _linear_relu_body(x, w, b, o):
    o[...] = jax.nn.relu(
        jnp.dot(x[...], w[...], preferred_element_type=_f32) + b[...])


def _tc_linear_relu(x, wT, b):
    return pl.pallas_call(
        _linear_relu_body,
        grid=(GRID,),
        in_specs=[
            pl.BlockSpec((BR, D), lambda i: (i, 0)),
            pl.BlockSpec((D, D), lambda i: (0, 0)),
            pl.BlockSpec((1, D), lambda i: (0, 0)),
        ],
        out_specs=pl.BlockSpec((BR, D), lambda i: (i, 0)),
        out_shape=jax.ShapeDtypeStruct((NP, D), _f32),
    )(x, wT, b)


def _mean_body(p, cnt, o):
    s = p[0] + p[1]
    c = cnt[0] + cnt[1]
    o[...] = s / jnp.maximum(c, 1.0)


def _tc_mean(parts, cnt_parts):
    return pl.pallas_call(
        _mean_body,
        grid=(GRID,),
        in_specs=[
            pl.BlockSpec((2, BR, D), lambda i: (0, i, 0)),
            pl.BlockSpec((2, BR, 1), lambda i: (0, i, 0)),
        ],
        out_specs=pl.BlockSpec((BR, D), lambda i: (i, 0)),
        out_shape=jax.ShapeDtypeStruct((NP, D), _f32),
    )(parts, cnt_parts)


def _layer_body(beta, p, h0, wT, o):
    xv = p[0] + p[1]
    xi = (1.0 - ALPHA) * xv + ALPHA * h0[...]
    o[...] = jax.nn.relu(
        (1.0 - beta) * xi
        + beta * jnp.dot(xi, wT[...], preferred_element_type=_f32))


def _tc_layer(parts, h0, wT, beta):
    return pl.pallas_call(
        functools.partial(_layer_body, beta),
        grid=(GRID,),
        in_specs=[
            pl.BlockSpec((2, BR, D), lambda i: (0, i, 0)),
            pl.BlockSpec((BR, D), lambda i: (i, 0)),
            pl.BlockSpec((D, D), lambda i: (0, 0)),
        ],
        out_specs=pl.BlockSpec((BR, D), lambda i: (i, 0)),
        out_shape=jax.ShapeDtypeStruct((NP, D), _f32),
    )(parts, h0, wT)


def _layer_final_body(beta, ncls, p, h0, wT, wo, b, o):
    xv = p[0] + p[1]
    xi = (1.0 - ALPHA) * xv + ALPHA * h0[...]
    h = jax.nn.relu(
        (1.0 - beta) * xi
        + beta * jnp.dot(xi, wT[...], preferred_element_type=_f32))
    z = jnp.dot(h, wo[...], preferred_element_type=_f32) + b[...]
    col = lax.broadcasted_iota(jnp.int32, (BR, D), 1)
    valid = col < ncls
    zm = jnp.where(valid, z, -1e30)
    m = jnp.max(zm, axis=1, keepdims=True)
    e = jnp.where(valid, jnp.exp(z - m), 0.0)
    ssum = jnp.sum(e, axis=1, keepdims=True)
    o[...] = z - m - jnp.log(ssum)


def _tc_layer_final(parts, h0, wT, beta, wo, b, ncls):
    return pl.pallas_call(
        functools.partial(_layer_final_body, beta, ncls),
        grid=(GRID,),
        in_specs=[
            pl.BlockSpec((2, BR, D), lambda i: (0, i, 0)),
            pl.BlockSpec((BR, D), lambda i: (i, 0)),
            pl.BlockSpec((D, D), lambda i: (0, 0)),
            pl.BlockSpec((D, D), lambda i: (0, 0)),
            pl.BlockSpec((1, D), lambda i: (0, 0)),
        ],
        out_specs=pl.BlockSpec((BR, D), lambda i: (i, 0)),
        out_shape=jax.ShapeDtypeStruct((NP, D), _f32),
    )(parts, h0, wT, wo, b)


def kernel(x, V, E, W0, b0, Wc0, Wc1, Wout, bout):
    V = V.astype(jnp.int32)
    E = E.astype(jnp.int32)
    ncls = Wout.shape[0]

    xp = jnp.zeros((NP, D), _f32).at[:N].set(x)
    z128 = jnp.zeros((NP, D), _f32)
    z1 = jnp.zeros((NP,), _f32)
    ones1 = jnp.ones((K,), _f32)

    pad = (N + (jnp.arange(NNZ_PAD - NNZ, dtype=jnp.int32) % (NP - N)))
    gp = jnp.concatenate([V, pad])
    sp = jnp.concatenate([E, pad])

    h = _tc_linear_relu(xp, W0.T, b0[None, :])
    h0 = h

    woutT = jnp.zeros((D, D), _f32).at[:, :ncls].set(Wout.T)
    bout_p = jnp.zeros((1, D), _f32).at[0, :ncls].set(bout)

    beta0 = math.log(LAMDA + 1.0)
    beta1 = math.log(LAMDA / 2.0 + 1.0)

    pe, cnt_flat = _sc_phase_cnt(h, gp, sp, z128, z1, ones1)
    cnt_parts = cnt_flat.reshape(2, NP, 1)
    xe = _tc_mean(pe.reshape(2, NP, D), cnt_parts)
    pv = _sc_phase(xe, sp, gp, z128)
    h = _tc_layer(pv.reshape(2, NP, D), h0, Wc0.T, beta0)

    pe = _sc_phase(h, gp, sp, z128)
    xe = _tc_mean(pe.reshape(2, NP, D), cnt_parts)
    pv = _sc_phase(xe, sp, gp, z128)
    out = _tc_layer_final(pv.reshape(2, NP, D), h0, Wc1.T, beta1,
                          woutT, bout_p, ncls)
    return out[:N, :ncls]


# speedup vs baseline: 12.9244x; 1.0110x over previous
import functools
import math

import jax
import jax.numpy as jnp
from jax import lax
from jax.experimental import pallas as pl
from jax.experimental.pallas import tpu as pltpu
from jax.experimental.pallas import tpu_sc as plsc

N = 10000
NE = 10000
NNZ = 320000
D = 128
ALPHA = 0.1
LAMDA = 0.5

NC = 2
NS = 16
NW = NC * NS
NP = 10240
RPT = NP // NS
K = 88
NNZ_PAD = 326656
NCHUNK = NNZ_PAD // K
CPW = NCHUNK // NW
NBUF = 4

_mesh = plsc.VectorSubcoreMesh(core_axis_name="c", subcore_axis_name="s")
_f32 = jnp.float32


def _sc_phase_body(with_count, *refs):
    if with_count:
        src, gidx, sidx, z128, z1, ones1h, out, cnt_out = refs[:8]
        acc, acc1 = refs[8], refs[9]
        rest, ptr = refs, 10
    else:
        src, gidx, sidx, z128, out = refs[:5]
        acc = refs[5]
        rest, ptr = refs, 6
    ig = rest[ptr:ptr + NBUF]; ptr += NBUF
    isc = rest[ptr:ptr + NBUF]; ptr += NBUF
    rows = rest[ptr:ptr + NBUF]; ptr += NBUF
    if with_count:
        ones1 = rest[ptr]; ptr += 1
    sgi = rest[ptr:ptr + NBUF]; ptr += NBUF
    ssi = rest[ptr:ptr + NBUF]; ptr += NBUF
    sr = rest[ptr:ptr + NBUF]; ptr += NBUF
    sw = rest[ptr:ptr + NBUF]; ptr += NBUF
    if with_count:
        sc_ = rest[ptr:ptr + NBUF]
    c = lax.axis_index("c")
    s = lax.axis_index("s")
    wid = s * NC + c
    row0 = s * RPT
    pltpu.async_copy(z128.at[pl.ds(row0, RPT)], acc.at[pl.ds(row0, RPT)], sr[0])
    if with_count:
        pltpu.async_copy(z1.at[pl.ds(row0, RPT)], acc1.at[pl.ds(row0, RPT)],
                         sr[1])
        pltpu.async_copy(ones1h, ones1, sr[2])

    base0 = pl.multiple_of(wid * CPW * K, 8)

    def idx_slice(h, j):
        return h.at[pl.ds(base0 + j * K, K)]

    for b in range(NBUF):
        pltpu.async_copy(idx_slice(gidx, b), ig[b], sgi[b])
        pltpu.async_copy(idx_slice(sidx, b), isc[b], ssi[b])
    pltpu.make_async_copy(z128.at[pl.ds(row0, RPT)], acc.at[pl.ds(row0, RPT)],
                          sr[0]).wait()
    if with_count:
        pltpu.make_async_copy(z1.at[pl.ds(row0, RPT)],
                              acc1.at[pl.ds(row0, RPT)], sr[1]).wait()
        pltpu.make_async_copy(ones1h, ones1, sr[2]).wait()
    plsc.subcore_barrier()
    for b in range(NBUF):
        pltpu.make_async_copy(idx_slice(gidx, b), ig[b], sgi[b]).wait()
        pltpu.async_copy(src.at[ig[b]], rows[b], sr[b])

    def body(t, carry):
        for b in range(NBUF):
            j = t * NBUF + b
            pltpu.make_async_copy(src.at[ig[b]], rows[b], sr[b]).wait()
            pltpu.async_copy(idx_slice(gidx, j + NBUF), ig[b], sgi[b])
            pltpu.make_async_copy(idx_slice(sidx, j), isc[b], ssi[b]).wait()
            pltpu.async_copy(rows[b], acc.at[isc[b]], sw[b], add=True)
            if with_count:
                pltpu.async_copy(ones1, acc1.at[isc[b]], sc_[b], add=True)
            pltpu.make_async_copy(rows[b], acc.at[isc[b]], sw[b]).wait()
            if with_count:
                pltpu.make_async_copy(ones1, acc1.at[isc[b]], sc_[b]).wait()
            pltpu.async_copy(idx_slice(sidx, j + NBUF), isc[b], ssi[b])
            pltpu.make_async_copy(idx_slice(gidx, j + NBUF), ig[b], sgi[b]).wait()
            pltpu.async_copy(src.at[ig[b]], rows[b], sr[b])
        return carry

    lax.fori_loop(0, CPW // NBUF - 1, body, 0)
    for b in range(NBUF):
        j = CPW - NBUF + b
        pltpu.make_async_copy(src.at[ig[b]], rows[b], sr[b]).wait()
        pltpu.make_async_copy(idx_slice(sidx, j), isc[b], ssi[b]).wait()
        pltpu.async_copy(rows[b], acc.at[isc[b]], sw[b], add=True)
        if with_count:
            pltpu.async_copy(ones1, acc1.at[isc[b]], sc_[b], add=True)
    for b in range(NBUF):
        pltpu.make_async_copy(rows[b], acc.at[isc[b]], sw[b]).wait()
        if with_count:
            pltpu.make_async_copy(ones1, acc1.at[isc[b]], sc_[b]).wait()
    plsc.subcore_barrier()
    obase = pl.multiple_of(c * NP + row0, 8)
    pltpu.sync_copy(acc.at[pl.ds(row0, RPT)], out.at[pl.ds(obase, RPT)])
    if with_count:
        pltpu.sync_copy(acc1.at[pl.ds(row0, RPT)], cnt_out.at[pl.ds(obase, RPT)])


def _make_phase(with_count):
    idx_bufs = [pltpu.VMEM((K,), jnp.int32)] * (2 * NBUF)
    row_bufs = [pltpu.VMEM((K, D), _f32)] * NBUF
    if with_count:
        out_type = (jax.ShapeDtypeStruct((2 * NP, D), _f32),
                    jax.ShapeDtypeStruct((2 * NP,), _f32))
        scratch = ([pltpu.VMEM_SHARED((NP, D), _f32),
                    pltpu.VMEM_SHARED((NP,), _f32)]
                   + idx_bufs + row_bufs + [pltpu.VMEM((K,), _f32)]
                   + [pltpu.SemaphoreType.DMA] * (5 * NBUF))
    else:
        out_type = jax.ShapeDtypeStruct((2 * NP, D), _f32)
        scratch = ([pltpu.VMEM_SHARED((NP, D), _f32)]
                   + idx_bufs + row_bufs
                   + [pltpu.SemaphoreType.DMA] * (4 * NBUF))
    return pl.kernel(
        functools.partial(_sc_phase_body, with_count),
        out_type=out_type,
        mesh=_mesh,
        scratch_types=scratch,
    )


_sc_phase_cnt = _make_phase(True)
_sc_phase = _make_phase(False)


BR = 1024
GRID = NP // BR


def _linear_relu_body(x, w, b, o):
    o[...] = jax.nn.relu(
        jnp.dot(x[...], w[...], preferred_element_type=_f32) + b[...])


def _tc_linear_relu(x, wT, b):
    return pl.pallas_call(
        _linear_relu_body,
        grid=(GRID,),
        in_specs=[
            pl.BlockSpec((BR, D), lambda i: (i, 0)),
            pl.BlockSpec((D, D), lambda i: (0, 0)),
            pl.BlockSpec((1, D), lambda i: (0, 0)),
        ],
        out_specs=pl.BlockSpec((BR, D), lambda i: (i, 0)),
        out_shape=jax.ShapeDtypeStruct((NP, D), _f32),
    )(x, wT, b)


def _mean_body(p, cnt, o):
    s = p[0] + p[1]
    c = cnt[0] + cnt[1]
    o[...] = s / jnp.maximum(c, 1.0)


def _tc_mean(parts, cnt_parts):
    return pl.pallas_call(
        _mean_body,
        grid=(GRID,),
        in_specs=[
            pl.BlockSpec((2, BR, D), lambda i: (0, i, 0)),
            pl.BlockSpec((2, BR, 1), lambda i: (0, i, 0)),
        ],
        out_specs=pl.BlockSpec((BR, D), lambda i: (i, 0)),
        out_shape=jax.ShapeDtypeStruct((NP, D), _f32),
    )(parts, cnt_parts)


def _layer_body(beta, p, h0, wT, o):
    xv = p[0] + p[1]
    xi = (1.0 - ALPHA) * xv + ALPHA * h0[...]
    o[...] = jax.nn.relu(
        (1.0 - beta) * xi
        + beta * jnp.dot(xi, wT[...], preferred_element_type=_f32))


def _tc_layer(parts, h0, wT, beta):
    return pl.pallas_call(
        functools.partial(_layer_body, beta),
        grid=(GRID,),
        in_specs=[
            pl.BlockSpec((2, BR, D), lambda i: (0, i, 0)),
            pl.BlockSpec((BR, D), lambda i: (i, 0)),
            pl.BlockSpec((D, D), lambda i: (0, 0)),
        ],
        out_specs=pl.BlockSpec((BR, D), lambda i: (i, 0)),
        out_shape=jax.ShapeDtypeStruct((NP, D), _f32),
    )(parts, h0, wT)


def _layer_final_body(beta, ncls, p, h0, wT, wo, b, o):
    xv = p[0] + p[1]
    xi = (1.0 - ALPHA) * xv + ALPHA * h0[...]
    h = jax.nn.relu(
        (1.0 - beta) * xi
        + beta * jnp.dot(xi, wT[...], preferred_element_type=_f32))
    z = jnp.dot(h, wo[...], preferred_element_type=_f32) + b[...]
    col = lax.broadcasted_iota(jnp.int32, (BR, D), 1)
    valid = col < ncls
    zm = jnp.where(valid, z, -1e30)
    m = jnp.max(zm, axis=1, keepdims=True)
    e = jnp.where(valid, jnp.exp(z - m), 0.0)
    ssum = jnp.sum(e, axis=1, keepdims=True)
    o[...] = z - m - jnp.log(ssum)


def _tc_layer_final(parts, h0, wT, beta, wo, b, ncls):
    return pl.pallas_call(
        functools.partial(_layer_final_body, beta, ncls),
        grid=(GRID,),
        in_specs=[
            pl.BlockSpec((2, BR, D), lambda i: (0, i, 0)),
            pl.BlockSpec((BR, D), lambda i: (i, 0)),
            pl.BlockSpec((D, D), lambda i: (0, 0)),
            pl.BlockSpec((D, D), lambda i: (0, 0)),
            pl.BlockSpec((1, D), lambda i: (0, 0)),
        ],
        out_specs=pl.BlockSpec((BR, D), lambda i: (i, 0)),
        out_shape=jax.ShapeDtypeStruct((NP, D), _f32),
    )(parts, h0, wT, wo, b)


def kernel(x, V, E, W0, b0, Wc0, Wc1, Wout, bout):
    V = V.astype(jnp.int32)
    E = E.astype(jnp.int32)
    ncls = Wout.shape[0]

    xp = jnp.zeros((NP, D), _f32).at[:N].set(x)
    z128 = jnp.zeros((NP, D), _f32)
    z1 = jnp.zeros((NP,), _f32)
    ones1 = jnp.ones((K,), _f32)

    pad = (N + (jnp.arange(NNZ_PAD - NNZ, dtype=jnp.int32) % (NP - N)))
    gp = jnp.concatenate([V, pad])
    sp = jnp.concatenate([E, pad])

    h = _tc_linear_relu(xp, W0.T, b0[None, :])
    h0 = h

    woutT = jnp.zeros((D, D), _f32).at[:, :ncls].set(Wout.T)
    bout_p = jnp.zeros((1, D), _f32).at[0, :ncls].set(bout)

    beta0 = math.log(LAMDA + 1.0)
    beta1 = math.log(LAMDA / 2.0 + 1.0)

    pe, cnt_flat = _sc_phase_cnt(h, gp, sp, z128, z1, ones1)
    cnt_parts = cnt_flat.reshape(2, NP, 1)
    xe = _tc_mean(pe.reshape(2, NP, D), cnt_parts)
    pv = _sc_phase(xe, sp, gp, z128)
    h = _tc_layer(pv.reshape(2, NP, D), h0, Wc0.T, beta0)

    pe = _sc_phase(h, gp, sp, z128)
    xe = _tc_mean(pe.reshape(2, NP, D), cnt_parts)
    pv = _sc_phase(xe, sp, gp, z128)
    out = _tc_layer_final(pv.reshape(2, NP, D), h0, Wc1.T, beta1,
                          woutT, bout_p, ncls)
    return out[:N, :ncls]
